# Initial kernel scaffold; baseline (speedup 1.0000x reference)
import jax
import jax.numpy as jnp
from jax.experimental import pallas as pl


def kernel(input_ids, wte):
    raise NotImplementedError("write your pallas kernel here")


# speedup vs baseline: 1.4413x; 1.4413x over previous
import functools

import jax
import jax.numpy as jnp
from jax import lax
from jax.experimental import pallas as pl
from jax.experimental.pallas import tpu as pltpu
from jax.experimental.pallas import tpu_sc as plsc

HIDDEN = 2048
NUM_WORKERS = 32


def _emb_body(bpw, chunk, nchunk, idx_hbm, tab_hbm, out_hbm, idx_v, rows_v, sem):
    wid = lax.axis_index("s") * 2 + lax.axis_index("c")
    base = wid * bpw
    pltpu.sync_copy(idx_hbm.at[pl.ds(base, bpw)], idx_v)

    def body(ci, _):
        off = ci * chunk
        pltpu.async_copy(
            tab_hbm.at[idx_v.at[pl.ds(off, chunk)]], rows_v, sem
        ).wait()
        pltpu.sync_copy(rows_v, out_hbm.at[pl.ds(base + off, chunk)])
        return 0

    lax.fori_loop(0, nchunk, body, 0)


def kernel(input_ids, wte):
    batch, seq = input_ids.shape
    b_total = batch * seq
    idx = input_ids.reshape(b_total).astype(jnp.int32)
    bpw = b_total // NUM_WORKERS
    chunk = 16
    nchunk = bpw // chunk

    mesh = plsc.VectorSubcoreMesh(core_axis_name="c", subcore_axis_name="s")
    emb = functools.partial(
        pl.kernel,
        mesh=mesh,
        out_type=jax.ShapeDtypeStruct((b_total, HIDDEN), jnp.float32),
        scratch_types=[
            pltpu.VMEM((bpw,), jnp.int32),
            pltpu.VMEM((chunk, HIDDEN), jnp.float32),
            pltpu.SemaphoreType.DMA,
        ],
    )(functools.partial(_emb_body, bpw, chunk, nchunk))

    out = emb(idx, wte)
    return out.reshape(batch, seq, HIDDEN)


# speedup vs baseline: 1.6256x; 1.1279x over previous
import functools

import jax
import jax.numpy as jnp
from jax import lax
from jax.experimental import pallas as pl
from jax.experimental.pallas import tpu as pltpu
from jax.experimental.pallas import tpu_sc as plsc

HIDDEN = 2048
NUM_WORKERS = 32
CHUNK = 16
NBUF = 2


def _emb_body(bpw, nchunk, idx_hbm, tab_hbm, out_hbm, idx_v, rows_v,
              gsem0, gsem1, ssem0, ssem1):
    gsems = (gsem0, gsem1)
    ssems = (ssem0, ssem1)
    wid = lax.axis_index("s") * 2 + lax.axis_index("c")
    base = wid * bpw
    pltpu.sync_copy(idx_hbm.at[pl.ds(base, bpw)], idx_v)

    def gather(c, b):
        return pltpu.async_copy(
            tab_hbm.at[idx_v.at[pl.ds(c * CHUNK, CHUNK)]],
            rows_v.at[b], gsems[b])

    def store(c, b):
        return pltpu.async_copy(
            rows_v.at[b],
            out_hbm.at[pl.ds(base + c * CHUNK, CHUNK)], ssems[b])

    gcopy = [gather(b, b) for b in range(NBUF)]
    scopy = [None] * NBUF
    for c in range(nchunk):
        b = c % NBUF
        gcopy[b].wait()
        scopy[b] = store(c, b)
        if c + NBUF < nchunk:
            scopy[b].wait()
            gcopy[b] = gather(c + NBUF, b)
    for c in range(nchunk - NBUF, nchunk):
        scopy[c % NBUF].wait()


def kernel(input_ids, wte):
    batch, seq = input_ids.shape
    b_total = batch * seq
    idx = input_ids.reshape(b_total).astype(jnp.int32)
    bpw = b_total // NUM_WORKERS
    nchunk = bpw // CHUNK

    mesh = plsc.VectorSubcoreMesh(core_axis_name="c", subcore_axis_name="s")
    emb = functools.partial(
        pl.kernel,
        mesh=mesh,
        out_type=jax.ShapeDtypeStruct((b_total, HIDDEN), jnp.float32),
        scratch_types=[
            pltpu.VMEM((bpw,), jnp.int32),
            pltpu.VMEM((NBUF, CHUNK, HIDDEN), jnp.float32),
            pltpu.SemaphoreType.DMA,
            pltpu.SemaphoreType.DMA,
            pltpu.SemaphoreType.DMA,
            pltpu.SemaphoreType.DMA,
        ],
    )(functools.partial(_emb_body, bpw, nchunk))

    out = emb(idx, wte)
    return out.reshape(batch, seq, HIDDEN)


# speedup vs baseline: 1.6531x; 1.0169x over previous
import functools

import jax
import jax.numpy as jnp
from jax import lax
from jax.experimental import pallas as pl
from jax.experimental.pallas import tpu as pltpu
from jax.experimental.pallas import tpu_sc as plsc

HIDDEN = 2048
NUM_WORKERS = 32
CHUNK = 16
NBUF = 3


def _emb_body(bpw, nchunk, idx_hbm, tab_hbm, out_hbm, idx_v, rows_v,
              gsem0, gsem1, gsem2, ssem0, ssem1, ssem2):
    gsems = (gsem0, gsem1, gsem2)
    ssems = (ssem0, ssem1, ssem2)
    wid = lax.axis_index("s") * 2 + lax.axis_index("c")
    base = wid * bpw
    pltpu.sync_copy(idx_hbm.at[pl.ds(base, bpw)], idx_v)

    def gather(c, b):
        return pltpu.async_copy(
            tab_hbm.at[idx_v.at[pl.ds(c * CHUNK, CHUNK)]],
            rows_v.at[b], gsems[b])

    def store(c, b):
        return pltpu.async_copy(
            rows_v.at[b],
            out_hbm.at[pl.ds(base + c * CHUNK, CHUNK)], ssems[b])

    gcopy = [gather(b, b) for b in range(NBUF)]
    scopy = [None] * NBUF
    for c in range(nchunk):
        b = c % NBUF
        gcopy[b].wait()
        scopy[b] = store(c, b)
        if c + NBUF < nchunk:
            scopy[b].wait()
            gcopy[b] = gather(c + NBUF, b)
    for c in range(nchunk - NBUF, nchunk):
        scopy[c % NBUF].wait()


def kernel(input_ids, wte):
    batch, seq = input_ids.shape
    b_total = batch * seq
    idx = input_ids.reshape(b_total).astype(jnp.int32)
    bpw = b_total // NUM_WORKERS
    nchunk = bpw // CHUNK

    mesh = plsc.VectorSubcoreMesh(core_axis_name="c", subcore_axis_name="s")
    emb = functools.partial(
        pl.kernel,
        mesh=mesh,
        out_type=jax.ShapeDtypeStruct((b_total, HIDDEN), jnp.float32),
        scratch_types=[
            pltpu.VMEM((bpw,), jnp.int32),
            pltpu.VMEM((NBUF, CHUNK, HIDDEN), jnp.float32),
            pltpu.SemaphoreType.DMA,
            pltpu.SemaphoreType.DMA,
            pltpu.SemaphoreType.DMA,
            pltpu.SemaphoreType.DMA,
            pltpu.SemaphoreType.DMA,
            pltpu.SemaphoreType.DMA,
        ],
    )(functools.partial(_emb_body, bpw, nchunk))

    out = emb(idx, wte)
    return out.reshape(batch, seq, HIDDEN)
